# baseline (speedup 1.0000x reference)
import jax
import jax.numpy as jnp
from jax import lax
from jax.experimental import pallas as pl
from jax.experimental.pallas import tpu as pltpu

N = 1024
DIN = 256
H = 8
DH = 32
HDH = H * DH
DE = DH + 1
DEP = 40
NEG_SLOPE = 0.2
ALPHA = 0.8
BETA = 0.5

BU = 256
NU = N // BU


def _leaky(x):
    return jnp.where(x > 0, x, NEG_SLOPE * x)


def _fused_kernel(feat_ref,
                  adj0_ref, adj1_ref, adj2_ref,
                  dg0_ref, dg1_ref, dg2_ref,
                  W0_ref, alm0_ref, arm0_ref, b0_ref,
                  W1_ref, alm1_ref, arm1_ref, b1_ref,
                  W2_ref, alm2_ref, arm2_ref, b2_ref,
                  qW_ref, qb_ref, kW_ref, kb_ref, mvWT_ref, mvb_ref,
                  mvT_ref, resT_ref,
                  rhs1_s, rhs2_s, elb_s, nerT_s, B1_s, B2_s, coefT_s,
                  numA_s, numB_s):
    ui = pl.program_id(0)
    adj_refs = (adj0_ref, adj1_ref, adj2_ref)
    dg_refs = (dg0_ref, dg1_ref, dg2_ref)
    W_refs = (W0_ref, W1_ref, W2_ref)
    alm_refs = (alm0_ref, alm1_ref, alm2_ref)
    arm_refs = (arm0_ref, arm1_ref, arm2_ref)
    b_refs = (b0_ref, b1_ref, b2_ref)
    dnT = (((0,), (1,)), ((), ()))
    dn0 = (((0,), (0,)), ((), ()))

    @pl.when(ui == 0)
    def _setup():
        feat = feat_ref[...]
        ones_col = jnp.ones((N, 1), dtype=jnp.float32)
        for v in range(3):
            h = jnp.dot(feat, W_refs[v][...], preferred_element_type=jnp.float32)
            el = jnp.dot(h, alm_refs[v][...], preferred_element_type=jnp.float32)
            elT = lax.dot_general(alm_refs[v][...], h, dnT,
                                  preferred_element_type=jnp.float32)
            erT = lax.dot_general(arm_refs[v][...], h, dnT,
                                  preferred_element_type=jnp.float32)
            elmax_row = jnp.max(el, axis=0, keepdims=True)
            elmax_col = jnp.max(elT, axis=1, keepdims=True)
            MT = _leaky(elmax_col + erT)
            A1 = jnp.exp(el - elmax_row)
            A2 = jnp.exp(NEG_SLOPE * (el - elmax_row))
            B1_s[v] = jnp.exp(erT + elmax_col - MT)
            B2_s[v] = jnp.exp(NEG_SLOPE * (erT + elmax_col) - MT)
            coefT_s[v] = ((1.0 - jnp.ceil(dg_refs[v][...]))
                          * jnp.exp(_leaky(elT + erT) - MT))
            elb_s[v] = el.astype(jnp.bfloat16)
            nerT_s[v] = (-erT).astype(jnp.bfloat16)
            for hh in range(H):
                hx = jnp.concatenate(
                    [h[:, hh * DH:(hh + 1) * DH], ones_col], axis=1)
                rhs1_s[v, hh, :, 0:DE] = (A1[:, hh:hh + 1] * hx).astype(jnp.bfloat16)
                rhs2_s[v, hh, :, 0:DE] = (A2[:, hh:hh + 1] * hx).astype(jnp.bfloat16)
        numA_s[...] = jnp.zeros_like(numA_s)
        numB_s[...] = jnp.zeros_like(numB_s)

    for v in range(3):
        adjc = jnp.ceil(adj_refs[v][...]).astype(jnp.bfloat16)
        zero = jnp.zeros_like(adjc)
        for hh in range(H):
            cond = elb_s[v, pl.ds(ui * BU, BU), hh:hh + 1] >= nerT_s[v, hh:hh + 1, :]
            m1 = jnp.where(cond, adjc, zero)
            m2 = adjc - m1
            numA_s[v, hh, 0:DE, :] += lax.dot_general(
                rhs1_s[v, hh, pl.ds(ui * BU, BU), 0:DE], m1, dn0,
                preferred_element_type=jnp.float32)
            numB_s[v, hh, 0:DE, :] += lax.dot_general(
                rhs2_s[v, hh, pl.ds(ui * BU, BU), 0:DE], m2, dn0,
                preferred_element_type=jnp.float32)

    @pl.when(ui == NU - 1)
    def _finish():
        feat = feat_ref[...]
        d_k = jnp.sqrt(jnp.float32(DH * N))
        qW = qW_ref[...]
        kW = kW_ref[...]
        mvWT = mvWT_ref[...]
        views = []
        logits = []
        gs = []
        for v in range(3):
            hT = lax.dot_general(W_refs[v][...], feat, dnT,
                                 preferred_element_type=jnp.float32)
            rows = []
            for hh in range(H):
                b1r = B1_s[v, hh:hh + 1, :]
                b2r = B2_s[v, hh:hh + 1, :]
                cfr = coefT_s[v, hh:hh + 1, :]
                hsl = hT[hh * DH:(hh + 1) * DH, :]
                num = (b1r * numA_s[v, hh, 0:DH, :]
                       + b2r * numB_s[v, hh, 0:DH, :] + cfr * hsl)
                den = (b1r * numA_s[v, hh, DH:DE, :]
                       + b2r * numB_s[v, hh, DH:DE, :] + cfr)
                o = num / den + b_refs[v][hh * DH:(hh + 1) * DH, :]
                rows.append(jnp.maximum(o, 0.0))
            sv = jnp.concatenate(rows, axis=0)
            views.append(sv)
            Qv = lax.dot_general(qW, sv, dn0,
                                 preferred_element_type=jnp.float32) + qb_ref[...]
            Kv = lax.dot_general(kW, sv, dn0,
                                 preferred_element_type=jnp.float32) + kb_ref[...]
            logits.append(jnp.sum(Qv * Kv) / d_k)
            gs.append(jnp.sum(sv * mvWT))

        m = jnp.maximum(jnp.maximum(logits[0], logits[1]), logits[2])
        ex = [jnp.exp(l - m) for l in logits]
        tot = ex[0] + ex[1] + ex[2]
        mvb = mvb_ref[0, 0]
        c = [ALPHA * (e / tot) + (1.0 - ALPHA) for e in ex]
        omega = [jax.nn.sigmoid(c[v] * gs[v] + mvb) for v in range(3)]
        mvT = (omega[0] * c[0] * views[0] + omega[1] * c[1] * views[1]
               + omega[2] * c[2] * views[2])
        mvT_ref[...] = mvT.T
        for v in range(3):
            resT_ref[v] = (BETA * c[v] * views[v] + (1.0 - BETA) * mvT).T


def _block_diag_attn(a):
    out = jnp.zeros((H, DH, H), dtype=a.dtype)
    out = out.at[jnp.arange(H), :, jnp.arange(H)].set(a)
    return out.reshape(HDH, H)


@jax.jit
def kernel(feature, s_adj, t_adj, poi_adj,
           sW, s_al, s_ar, s_b,
           tW, t_al, t_ar, t_b,
           pW, p_al, p_ar, p_b,
           qW, qb, kW, kb, mvW, mvb):
    full = lambda *shape: pl.BlockSpec(shape, lambda ui: (0,) * len(shape))
    per_view_specs = []
    for _ in range(3):
        per_view_specs += [
            full(DIN, HDH),
            full(HDH, H),
            full(HDH, H),
            full(HDH, 1),
        ]

    mv_out, result = pl.pallas_call(
        _fused_kernel,
        grid=(NU,),
        in_specs=[
            full(N, DIN),
            pl.BlockSpec((BU, N), lambda ui: (ui, 0)),
            pl.BlockSpec((BU, N), lambda ui: (ui, 0)),
            pl.BlockSpec((BU, N), lambda ui: (ui, 0)),
            full(1, N),
            full(1, N),
            full(1, N),
            *per_view_specs,
            full(HDH, DH),
            full(DH, 1),
            full(HDH, DH),
            full(DH, 1),
            full(HDH, N),
            full(1, 1),
        ],
        out_specs=[
            full(N, HDH),
            full(3, N, HDH),
        ],
        out_shape=[
            jax.ShapeDtypeStruct((N, HDH), jnp.float32),
            jax.ShapeDtypeStruct((3, N, HDH), jnp.float32),
        ],
        scratch_shapes=[
            pltpu.VMEM((3, H, N, DEP), jnp.bfloat16),
            pltpu.VMEM((3, H, N, DEP), jnp.bfloat16),
            pltpu.VMEM((3, N, H), jnp.bfloat16),
            pltpu.VMEM((3, H, N), jnp.bfloat16),
            pltpu.VMEM((3, H, N), jnp.float32),
            pltpu.VMEM((3, H, N), jnp.float32),
            pltpu.VMEM((3, H, N), jnp.float32),
            pltpu.VMEM((3, H, DEP, N), jnp.float32),
            pltpu.VMEM((3, H, DEP, N), jnp.float32),
        ],
    )(feature, s_adj, t_adj, poi_adj,
      jnp.diagonal(s_adj).reshape(1, N), jnp.diagonal(t_adj).reshape(1, N),
      jnp.diagonal(poi_adj).reshape(1, N),
      sW, _block_diag_attn(s_al), _block_diag_attn(s_ar), s_b.reshape(HDH, 1),
      tW, _block_diag_attn(t_al), _block_diag_attn(t_ar), t_b.reshape(HDH, 1),
      pW, _block_diag_attn(p_al), _block_diag_attn(p_ar), p_b.reshape(HDH, 1),
      qW, qb.reshape(DH, 1), kW, kb.reshape(DH, 1),
      mvW.reshape(N, HDH).T, mvb.reshape(1, 1))

    return (mv_out, result)


# speedup vs baseline: 1.0033x; 1.0033x over previous
import jax
import jax.numpy as jnp
from jax import lax
from jax.experimental import pallas as pl
from jax.experimental.pallas import tpu as pltpu

N = 1024
DIN = 256
H = 8
DH = 32
HDH = H * DH
DE = DH + 1
DEP = 40
NEG_SLOPE = 0.2
ALPHA = 0.8
BETA = 0.5

BU = 256
NU = N // BU


def _leaky(x):
    return jnp.where(x > 0, x, NEG_SLOPE * x)


def _fused_kernel(feat_ref,
                  adj0_ref, adj1_ref, adj2_ref,
                  dg0_ref, dg1_ref, dg2_ref,
                  W0_ref, alm0_ref, arm0_ref, b0_ref,
                  W1_ref, alm1_ref, arm1_ref, b1_ref,
                  W2_ref, alm2_ref, arm2_ref, b2_ref,
                  qW_ref, qb_ref, kW_ref, kb_ref, mvWT_ref, mvb_ref,
                  mvT_ref, resT_ref,
                  rhs1_s, rhs2_s, elb_s, nerT_s, B1_s, B2_s, coefT_s,
                  numA_s, numB_s):
    ui = pl.program_id(0)
    adj_refs = (adj0_ref, adj1_ref, adj2_ref)
    dg_refs = (dg0_ref, dg1_ref, dg2_ref)
    W_refs = (W0_ref, W1_ref, W2_ref)
    alm_refs = (alm0_ref, alm1_ref, alm2_ref)
    arm_refs = (arm0_ref, arm1_ref, arm2_ref)
    b_refs = (b0_ref, b1_ref, b2_ref)
    dnT = (((0,), (1,)), ((), ()))
    dn0 = (((0,), (0,)), ((), ()))

    @pl.when(ui == 0)
    def _setup():
        feat = feat_ref[...]
        ones_col = jnp.ones((N, 1), dtype=jnp.float32)
        for v in range(3):
            h = jnp.dot(feat, W_refs[v][...], preferred_element_type=jnp.float32)
            el = jnp.dot(h, alm_refs[v][...], preferred_element_type=jnp.float32)
            elT = lax.dot_general(alm_refs[v][...], h, dnT,
                                  preferred_element_type=jnp.float32)
            erT = lax.dot_general(arm_refs[v][...], h, dnT,
                                  preferred_element_type=jnp.float32)
            elmax_row = jnp.max(el, axis=0, keepdims=True)
            elmax_col = jnp.max(elT, axis=1, keepdims=True)
            MT = _leaky(elmax_col + erT)
            A1 = jnp.exp(el - elmax_row)
            A2 = jnp.exp(NEG_SLOPE * (el - elmax_row))
            B1_s[v] = jnp.exp(erT + elmax_col - MT)
            B2_s[v] = jnp.exp(NEG_SLOPE * (erT + elmax_col) - MT)
            coefT_s[v] = ((1.0 - jnp.ceil(dg_refs[v][...]))
                          * jnp.exp(_leaky(elT + erT) - MT))
            for uu in range(NU):
                sl = slice(uu * BU, (uu + 1) * BU)
                elb_s[v, uu] = el[sl, :].astype(jnp.bfloat16)
            nerT_s[v] = (-erT).astype(jnp.bfloat16)
            zpad = jnp.zeros((N, DEP - DE), dtype=jnp.bfloat16)
            for hh in range(H):
                hx = jnp.concatenate(
                    [h[:, hh * DH:(hh + 1) * DH], ones_col], axis=1)
                r1 = jnp.concatenate(
                    [(A1[:, hh:hh + 1] * hx).astype(jnp.bfloat16), zpad], axis=1)
                r2 = jnp.concatenate(
                    [(A2[:, hh:hh + 1] * hx).astype(jnp.bfloat16), zpad], axis=1)
                for uu in range(NU):
                    sl = slice(uu * BU, (uu + 1) * BU)
                    rhs1_s[v, hh, uu] = r1[sl, :]
                    rhs2_s[v, hh, uu] = r2[sl, :]
        numA_s[...] = jnp.zeros_like(numA_s)
        numB_s[...] = jnp.zeros_like(numB_s)

    for v in range(3):
        adjc = jnp.ceil(adj_refs[v][...]).astype(jnp.bfloat16)
        zero = jnp.zeros_like(adjc)
        for hh in range(H):
            cond = elb_s[v, ui, :, hh:hh + 1] >= nerT_s[v, hh:hh + 1, :]
            m1 = jnp.where(cond, adjc, zero)
            m2 = adjc - m1
            numA_s[v, hh] += lax.dot_general(
                rhs1_s[v, hh, ui], m1, dn0,
                preferred_element_type=jnp.float32)
            numB_s[v, hh] += lax.dot_general(
                rhs2_s[v, hh, ui], m2, dn0,
                preferred_element_type=jnp.float32)

    @pl.when(ui == NU - 1)
    def _finish():
        feat = feat_ref[...]
        d_k = jnp.sqrt(jnp.float32(DH * N))
        qW = qW_ref[...]
        kW = kW_ref[...]
        mvWT = mvWT_ref[...]
        views = []
        logits = []
        gs = []
        for v in range(3):
            hT = lax.dot_general(W_refs[v][...], feat, dnT,
                                 preferred_element_type=jnp.float32)
            rows = []
            for hh in range(H):
                b1r = B1_s[v, hh:hh + 1, :]
                b2r = B2_s[v, hh:hh + 1, :]
                cfr = coefT_s[v, hh:hh + 1, :]
                hsl = hT[hh * DH:(hh + 1) * DH, :]
                num = (b1r * numA_s[v, hh, 0:DH, :]
                       + b2r * numB_s[v, hh, 0:DH, :] + cfr * hsl)
                den = (b1r * numA_s[v, hh, DH:DE, :]
                       + b2r * numB_s[v, hh, DH:DE, :] + cfr)
                o = num / den + b_refs[v][hh * DH:(hh + 1) * DH, :]
                rows.append(jnp.maximum(o, 0.0))
            sv = jnp.concatenate(rows, axis=0)
            views.append(sv)
            Qv = lax.dot_general(qW, sv, dn0,
                                 preferred_element_type=jnp.float32) + qb_ref[...]
            Kv = lax.dot_general(kW, sv, dn0,
                                 preferred_element_type=jnp.float32) + kb_ref[...]
            logits.append(jnp.sum(Qv * Kv) / d_k)
            gs.append(jnp.sum(sv * mvWT))

        m = jnp.maximum(jnp.maximum(logits[0], logits[1]), logits[2])
        ex = [jnp.exp(l - m) for l in logits]
        tot = ex[0] + ex[1] + ex[2]
        mvb = mvb_ref[0, 0]
        c = [ALPHA * (e / tot) + (1.0 - ALPHA) for e in ex]
        omega = [jax.nn.sigmoid(c[v] * gs[v] + mvb) for v in range(3)]
        mvT = (omega[0] * c[0] * views[0] + omega[1] * c[1] * views[1]
               + omega[2] * c[2] * views[2])
        mvT_ref[...] = mvT.T
        for v in range(3):
            resT_ref[v] = (BETA * c[v] * views[v] + (1.0 - BETA) * mvT).T


def _block_diag_attn(a):
    out = jnp.zeros((H, DH, H), dtype=a.dtype)
    out = out.at[jnp.arange(H), :, jnp.arange(H)].set(a)
    return out.reshape(HDH, H)


@jax.jit
def kernel(feature, s_adj, t_adj, poi_adj,
           sW, s_al, s_ar, s_b,
           tW, t_al, t_ar, t_b,
           pW, p_al, p_ar, p_b,
           qW, qb, kW, kb, mvW, mvb):
    full = lambda *shape: pl.BlockSpec(shape, lambda ui: (0,) * len(shape))
    per_view_specs = []
    for _ in range(3):
        per_view_specs += [
            full(DIN, HDH),
            full(HDH, H),
            full(HDH, H),
            full(HDH, 1),
        ]

    mv_out, result = pl.pallas_call(
        _fused_kernel,
        grid=(NU,),
        in_specs=[
            full(N, DIN),
            pl.BlockSpec((BU, N), lambda ui: (ui, 0)),
            pl.BlockSpec((BU, N), lambda ui: (ui, 0)),
            pl.BlockSpec((BU, N), lambda ui: (ui, 0)),
            full(1, N),
            full(1, N),
            full(1, N),
            *per_view_specs,
            full(HDH, DH),
            full(DH, 1),
            full(HDH, DH),
            full(DH, 1),
            full(HDH, N),
            full(1, 1),
        ],
        out_specs=[
            full(N, HDH),
            full(3, N, HDH),
        ],
        out_shape=[
            jax.ShapeDtypeStruct((N, HDH), jnp.float32),
            jax.ShapeDtypeStruct((3, N, HDH), jnp.float32),
        ],
        scratch_shapes=[
            pltpu.VMEM((3, H, NU, BU, DEP), jnp.bfloat16),
            pltpu.VMEM((3, H, NU, BU, DEP), jnp.bfloat16),
            pltpu.VMEM((3, NU, BU, H), jnp.bfloat16),
            pltpu.VMEM((3, H, N), jnp.bfloat16),
            pltpu.VMEM((3, H, N), jnp.float32),
            pltpu.VMEM((3, H, N), jnp.float32),
            pltpu.VMEM((3, H, N), jnp.float32),
            pltpu.VMEM((3, H, DEP, N), jnp.float32),
            pltpu.VMEM((3, H, DEP, N), jnp.float32),
        ],
    )(feature, s_adj, t_adj, poi_adj,
      jnp.diagonal(s_adj).reshape(1, N), jnp.diagonal(t_adj).reshape(1, N),
      jnp.diagonal(poi_adj).reshape(1, N),
      sW, _block_diag_attn(s_al), _block_diag_attn(s_ar), s_b.reshape(HDH, 1),
      tW, _block_diag_attn(t_al), _block_diag_attn(t_ar), t_b.reshape(HDH, 1),
      pW, _block_diag_attn(p_al), _block_diag_attn(p_ar), p_b.reshape(HDH, 1),
      qW, qb.reshape(DH, 1), kW, kb.reshape(DH, 1),
      mvW.reshape(N, HDH).T, mvb.reshape(1, 1))

    return (mv_out, result)


# speedup vs baseline: 1.7617x; 1.7558x over previous
import jax
import jax.numpy as jnp
from jax import lax
from jax.experimental import pallas as pl
from jax.experimental.pallas import tpu as pltpu

N = 1024
DIN = 256
H = 8
DH = 32
HDH = H * DH
DE = DH + 1
DEP = 40
NEG_SLOPE = 0.2
ALPHA = 0.8
BETA = 0.5

BU = 256
NU = N // BU


def _leaky(x):
    return jnp.where(x > 0, x, NEG_SLOPE * x)


def _fused_kernel(feat_ref,
                  adj0_ref, adj1_ref, adj2_ref,
                  W0_ref, alm0_ref, arm0_ref, b0_ref,
                  W1_ref, alm1_ref, arm1_ref, b1_ref,
                  W2_ref, alm2_ref, arm2_ref, b2_ref,
                  qW_ref, qb_ref, kW_ref, kb_ref, mvWT_ref, mvb_ref,
                  mvT_ref, resT_ref,
                  rhs1_s, rhs2_s, elb_s, nerT_s, B1_s, B2_s, coefT_s,
                  numA_s, numB_s):
    ui = pl.program_id(0)
    adj_refs = (adj0_ref, adj1_ref, adj2_ref)
    W_refs = (W0_ref, W1_ref, W2_ref)
    alm_refs = (alm0_ref, alm1_ref, alm2_ref)
    arm_refs = (arm0_ref, arm1_ref, arm2_ref)
    b_refs = (b0_ref, b1_ref, b2_ref)
    dnT = (((0,), (1,)), ((), ()))
    dn0 = (((0,), (0,)), ((), ()))

    @pl.when(ui == 0)
    def _setup():
        feat = feat_ref[...]
        ones_col = jnp.ones((N, 1), dtype=jnp.float32)
        for v in range(3):
            h = jnp.dot(feat, W_refs[v][...], preferred_element_type=jnp.float32)
            el = jnp.dot(h, alm_refs[v][...], preferred_element_type=jnp.float32)
            elT = lax.dot_general(alm_refs[v][...], h, dnT,
                                  preferred_element_type=jnp.float32)
            erT = lax.dot_general(arm_refs[v][...], h, dnT,
                                  preferred_element_type=jnp.float32)
            elmax_row = jnp.max(el, axis=0, keepdims=True)
            elmax_col = jnp.max(elT, axis=1, keepdims=True)
            MT = _leaky(elmax_col + erT)
            A1 = jnp.exp(el - elmax_row)
            A2 = jnp.exp(NEG_SLOPE * (el - elmax_row))
            B1_s[v] = jnp.exp(erT + elmax_col - MT)
            B2_s[v] = jnp.exp(NEG_SLOPE * (erT + elmax_col) - MT)
            coefT_s[v] = jnp.exp(_leaky(elT + erT) - MT)
            for uu in range(NU):
                sl = slice(uu * BU, (uu + 1) * BU)
                elb_s[v, uu] = el[sl, :].astype(jnp.bfloat16)
            nerT_s[v] = (-erT).astype(jnp.bfloat16)
            zpad = jnp.zeros((N, DEP - DE), dtype=jnp.bfloat16)
            for hh in range(H):
                hx = jnp.concatenate(
                    [h[:, hh * DH:(hh + 1) * DH], ones_col], axis=1)
                r1 = jnp.concatenate(
                    [(A1[:, hh:hh + 1] * hx).astype(jnp.bfloat16), zpad], axis=1)
                r2 = jnp.concatenate(
                    [(A2[:, hh:hh + 1] * hx).astype(jnp.bfloat16), zpad], axis=1)
                for uu in range(NU):
                    sl = slice(uu * BU, (uu + 1) * BU)
                    rhs1_s[v, hh, uu] = r1[sl, :]
                    rhs2_s[v, hh, uu] = r2[sl, :]
        numA_s[...] = jnp.zeros_like(numA_s)
        numB_s[...] = jnp.zeros_like(numB_s)

    rows = lax.broadcasted_iota(jnp.int32, (BU, N), 0) + ui * BU
    cols = lax.broadcasted_iota(jnp.int32, (BU, N), 1)
    offdiag = rows != cols
    for v in range(3):
        adjc = jnp.where(offdiag, jnp.ceil(adj_refs[v][...]),
                         0.0).astype(jnp.bfloat16)
        zero = jnp.zeros_like(adjc)
        for hh in range(H):
            cond = elb_s[v, ui, :, hh:hh + 1] >= nerT_s[v, hh:hh + 1, :]
            m1 = jnp.where(cond, adjc, zero)
            m2 = adjc - m1
            numA_s[v, hh] += lax.dot_general(
                rhs1_s[v, hh, ui], m1, dn0,
                preferred_element_type=jnp.float32)
            numB_s[v, hh] += lax.dot_general(
                rhs2_s[v, hh, ui], m2, dn0,
                preferred_element_type=jnp.float32)

    @pl.when(ui == NU - 1)
    def _finish():
        feat = feat_ref[...]
        d_k = jnp.sqrt(jnp.float32(DH * N))
        qW = qW_ref[...]
        kW = kW_ref[...]
        mvWT = mvWT_ref[...]
        views = []
        logits = []
        gs = []
        for v in range(3):
            hT = lax.dot_general(W_refs[v][...], feat, dnT,
                                 preferred_element_type=jnp.float32)
            rows = []
            for hh in range(H):
                b1r = B1_s[v, hh:hh + 1, :]
                b2r = B2_s[v, hh:hh + 1, :]
                cfr = coefT_s[v, hh:hh + 1, :]
                hsl = hT[hh * DH:(hh + 1) * DH, :]
                num = (b1r * numA_s[v, hh, 0:DH, :]
                       + b2r * numB_s[v, hh, 0:DH, :] + cfr * hsl)
                den = (b1r * numA_s[v, hh, DH:DE, :]
                       + b2r * numB_s[v, hh, DH:DE, :] + cfr)
                o = num / den + b_refs[v][hh * DH:(hh + 1) * DH, :]
                rows.append(jnp.maximum(o, 0.0))
            sv = jnp.concatenate(rows, axis=0)
            views.append(sv)
            Qv = lax.dot_general(qW, sv, dn0,
                                 preferred_element_type=jnp.float32) + qb_ref[...]
            Kv = lax.dot_general(kW, sv, dn0,
                                 preferred_element_type=jnp.float32) + kb_ref[...]
            logits.append(jnp.sum(Qv * Kv) / d_k)
            gs.append(jnp.sum(sv * mvWT))

        m = jnp.maximum(jnp.maximum(logits[0], logits[1]), logits[2])
        ex = [jnp.exp(l - m) for l in logits]
        tot = ex[0] + ex[1] + ex[2]
        mvb = mvb_ref[0, 0]
        c = [ALPHA * (e / tot) + (1.0 - ALPHA) for e in ex]
        omega = [jax.nn.sigmoid(c[v] * gs[v] + mvb) for v in range(3)]
        mvT = (omega[0] * c[0] * views[0] + omega[1] * c[1] * views[1]
               + omega[2] * c[2] * views[2])
        mvT_ref[...] = mvT.T
        for v in range(3):
            resT_ref[v] = (BETA * c[v] * views[v] + (1.0 - BETA) * mvT).T


def _block_diag_attn(a):
    out = jnp.zeros((H, DH, H), dtype=a.dtype)
    out = out.at[jnp.arange(H), :, jnp.arange(H)].set(a)
    return out.reshape(HDH, H)


@jax.jit
def kernel(feature, s_adj, t_adj, poi_adj,
           sW, s_al, s_ar, s_b,
           tW, t_al, t_ar, t_b,
           pW, p_al, p_ar, p_b,
           qW, qb, kW, kb, mvW, mvb):
    full = lambda *shape: pl.BlockSpec(shape, lambda ui: (0,) * len(shape))
    per_view_specs = []
    for _ in range(3):
        per_view_specs += [
            full(DIN, HDH),
            full(HDH, H),
            full(HDH, H),
            full(HDH, 1),
        ]

    mv_out, result = pl.pallas_call(
        _fused_kernel,
        grid=(NU,),
        in_specs=[
            full(N, DIN),
            pl.BlockSpec((BU, N), lambda ui: (ui, 0)),
            pl.BlockSpec((BU, N), lambda ui: (ui, 0)),
            pl.BlockSpec((BU, N), lambda ui: (ui, 0)),
            *per_view_specs,
            full(HDH, DH),
            full(DH, 1),
            full(HDH, DH),
            full(DH, 1),
            full(HDH, N),
            full(1, 1),
        ],
        out_specs=[
            full(N, HDH),
            full(3, N, HDH),
        ],
        out_shape=[
            jax.ShapeDtypeStruct((N, HDH), jnp.float32),
            jax.ShapeDtypeStruct((3, N, HDH), jnp.float32),
        ],
        scratch_shapes=[
            pltpu.VMEM((3, H, NU, BU, DEP), jnp.bfloat16),
            pltpu.VMEM((3, H, NU, BU, DEP), jnp.bfloat16),
            pltpu.VMEM((3, NU, BU, H), jnp.bfloat16),
            pltpu.VMEM((3, H, N), jnp.bfloat16),
            pltpu.VMEM((3, H, N), jnp.float32),
            pltpu.VMEM((3, H, N), jnp.float32),
            pltpu.VMEM((3, H, N), jnp.float32),
            pltpu.VMEM((3, H, DEP, N), jnp.float32),
            pltpu.VMEM((3, H, DEP, N), jnp.float32),
        ],
    )(feature, s_adj, t_adj, poi_adj,
      sW, _block_diag_attn(s_al), _block_diag_attn(s_ar), s_b.reshape(HDH, 1),
      tW, _block_diag_attn(t_al), _block_diag_attn(t_ar), t_b.reshape(HDH, 1),
      pW, _block_diag_attn(p_al), _block_diag_attn(p_ar), p_b.reshape(HDH, 1),
      qW, qb.reshape(DH, 1), kW, kb.reshape(DH, 1),
      mvW.reshape(N, HDH).T, mvb.reshape(1, 1))

    return (mv_out, result)


# speedup vs baseline: 1.9334x; 1.0975x over previous
import jax
import jax.numpy as jnp
import numpy as np
from jax import lax
from jax.experimental import pallas as pl
from jax.experimental.pallas import tpu as pltpu

N = 1024
DIN = 256
H = 8
DH = 32
HDH = H * DH
DE = DH + 1
DEP = 40
NEG_SLOPE = 0.2
ALPHA = 0.8
BETA = 0.5

BU = 256
NU = N // BU


def _leaky(x):
    return jnp.where(x > 0, x, NEG_SLOPE * x)


def _fused_kernel(feat_ref,
                  adj0_ref, adj1_ref, adj2_ref,
                  W0_ref, alm0_ref, arm0_ref, b0_ref,
                  W1_ref, alm1_ref, arm1_ref, b1_ref,
                  W2_ref, alm2_ref, arm2_ref, b2_ref,
                  qW_ref, qb_ref, kW_ref, kb_ref, mvWT_ref, mvb_ref,
                  mvT_ref, resT_ref,
                  rhs1_s, rhs2_s, elb_s, nerT_s, B1_s, B2_s, coefT_s,
                  numA_s, numB_s):
    ui = pl.program_id(0)
    adj_refs = (adj0_ref, adj1_ref, adj2_ref)
    W_refs = (W0_ref, W1_ref, W2_ref)
    alm_refs = (alm0_ref, alm1_ref, alm2_ref)
    arm_refs = (arm0_ref, arm1_ref, arm2_ref)
    b_refs = (b0_ref, b1_ref, b2_ref)
    dnT = (((0,), (1,)), ((), ()))
    dn0 = (((0,), (0,)), ((), ()))

    @pl.when(ui == 0)
    def _setup():
        feat = feat_ref[...]
        ones_col = jnp.ones((N, 1), dtype=jnp.float32)
        for v in range(3):
            h = jnp.dot(feat, W_refs[v][...], preferred_element_type=jnp.float32)
            el = jnp.dot(h, alm_refs[v][...], preferred_element_type=jnp.float32)
            elT = lax.dot_general(alm_refs[v][...], h, dnT,
                                  preferred_element_type=jnp.float32)
            erT = lax.dot_general(arm_refs[v][...], h, dnT,
                                  preferred_element_type=jnp.float32)
            elmax_row = jnp.max(el, axis=0, keepdims=True)
            elmax_col = jnp.max(elT, axis=1, keepdims=True)
            MT = _leaky(elmax_col + erT)
            A1 = jnp.exp(el - elmax_row)
            A2 = jnp.exp(NEG_SLOPE * (el - elmax_row))
            B1_s[v] = jnp.exp(erT + elmax_col - MT)
            B2_s[v] = jnp.exp(NEG_SLOPE * (erT + elmax_col) - MT)
            coefT_s[v] = jnp.exp(_leaky(elT + erT) - MT)
            for uu in range(NU):
                sl = slice(uu * BU, (uu + 1) * BU)
                elb_s[v, uu] = el[sl, :].astype(jnp.bfloat16)
            nerT_s[v] = (-erT).astype(jnp.bfloat16)
            zpad = jnp.zeros((N, DEP - DE), dtype=jnp.bfloat16)
            for hh in range(H):
                hx = jnp.concatenate(
                    [h[:, hh * DH:(hh + 1) * DH], ones_col], axis=1)
                r1 = jnp.concatenate(
                    [(A1[:, hh:hh + 1] * hx).astype(jnp.bfloat16), zpad], axis=1)
                r2 = jnp.concatenate(
                    [(A2[:, hh:hh + 1] * hx).astype(jnp.bfloat16), zpad], axis=1)
                for uu in range(NU):
                    sl = slice(uu * BU, (uu + 1) * BU)
                    rhs1_s[v, hh, uu] = r1[sl, :]
                    rhs2_s[v, hh, uu] = r2[sl, :]
        numA_s[...] = jnp.zeros_like(numA_s)
        numB_s[...] = jnp.zeros_like(numB_s)

    rows = lax.broadcasted_iota(jnp.int32, (BU, N), 0) + ui * BU
    cols = lax.broadcasted_iota(jnp.int32, (BU, N), 1)
    offdiag = rows != cols
    for v in range(3):
        adjc = jnp.where(offdiag, jnp.ceil(adj_refs[v][...]),
                         0.0).astype(jnp.bfloat16)
        zero = jnp.zeros_like(adjc)
        for hh in range(H):
            cond = elb_s[v, ui, :, hh:hh + 1] >= nerT_s[v, hh:hh + 1, :]
            m1 = jnp.where(cond, adjc, zero)
            m2 = adjc - m1
            numA_s[v, hh] += lax.dot_general(
                rhs1_s[v, hh, ui], m1, dn0,
                preferred_element_type=jnp.float32)
            numB_s[v, hh] += lax.dot_general(
                rhs2_s[v, hh, ui], m2, dn0,
                preferred_element_type=jnp.float32)

    @pl.when(ui == NU - 1)
    def _finish():
        feat = feat_ref[...]
        d_k = jnp.sqrt(jnp.float32(DH * N))
        qW = qW_ref[...]
        kW = kW_ref[...]
        mvW = mvWT_ref[...]
        diag_mask = (lax.broadcasted_iota(jnp.int32, (DH, DH), 0)
                     == lax.broadcasted_iota(jnp.int32, (DH, DH), 1)
                     ).astype(jnp.float32)
        views = []
        logits = []
        gs = []
        for v in range(3):
            hT = lax.dot_general(W_refs[v][...], feat, dnT,
                                 preferred_element_type=jnp.float32)
            rows = []
            for hh in range(H):
                b1r = B1_s[v, hh:hh + 1, :]
                b2r = B2_s[v, hh:hh + 1, :]
                cfr = coefT_s[v, hh:hh + 1, :]
                hsl = hT[hh * DH:(hh + 1) * DH, :]
                num = (b1r * numA_s[v, hh, 0:DH, :]
                       + b2r * numB_s[v, hh, 0:DH, :] + cfr * hsl)
                den = (b1r * numA_s[v, hh, DH:DE, :]
                       + b2r * numB_s[v, hh, DH:DE, :] + cfr)
                o = num / den + b_refs[v][hh * DH:(hh + 1) * DH, :]
                rows.append(jnp.maximum(o, 0.0))
            sv = jnp.concatenate(rows, axis=0)
            views.append(sv)
            Qv = lax.dot_general(qW, sv, dn0,
                                 preferred_element_type=jnp.float32) + qb_ref[...]
            Kv = lax.dot_general(kW, sv, dn0,
                                 preferred_element_type=jnp.float32) + kb_ref[...]
            logits.append(jnp.sum(Qv * Kv) / d_k)
            tr = jnp.float32(0.0)
            for hh in range(H):
                blk = lax.dot_general(
                    sv[hh * DH:(hh + 1) * DH, :], mvW[:, hh * DH:(hh + 1) * DH],
                    (((1,), (0,)), ((), ())), preferred_element_type=jnp.float32)
                tr = tr + jnp.sum(blk * diag_mask)
            gs.append(tr)

        m = jnp.maximum(jnp.maximum(logits[0], logits[1]), logits[2])
        ex = [jnp.exp(l - m) for l in logits]
        tot = ex[0] + ex[1] + ex[2]
        mvb = mvb_ref[0, 0]
        c = [ALPHA * (e / tot) + (1.0 - ALPHA) for e in ex]
        omega = [jax.nn.sigmoid(c[v] * gs[v] + mvb) for v in range(3)]
        mvT = (omega[0] * c[0] * views[0] + omega[1] * c[1] * views[1]
               + omega[2] * c[2] * views[2])
        mvT_ref[...] = mvT.T
        for v in range(3):
            resT_ref[v] = (BETA * c[v] * views[v] + (1.0 - BETA) * mvT).T


_BD_MASK = np.repeat(np.eye(H, dtype=np.float32), DH, axis=0)


def _block_diag_attn(a):
    return a.reshape(HDH, 1) * _BD_MASK


@jax.jit
def kernel(feature, s_adj, t_adj, poi_adj,
           sW, s_al, s_ar, s_b,
           tW, t_al, t_ar, t_b,
           pW, p_al, p_ar, p_b,
           qW, qb, kW, kb, mvW, mvb):
    full = lambda *shape: pl.BlockSpec(shape, lambda ui: (0,) * len(shape))
    per_view_specs = []
    for _ in range(3):
        per_view_specs += [
            full(DIN, HDH),
            full(HDH, H),
            full(HDH, H),
            full(HDH, 1),
        ]

    mv_out, result = pl.pallas_call(
        _fused_kernel,
        grid=(NU,),
        in_specs=[
            full(N, DIN),
            pl.BlockSpec((BU, N), lambda ui: (ui, 0)),
            pl.BlockSpec((BU, N), lambda ui: (ui, 0)),
            pl.BlockSpec((BU, N), lambda ui: (ui, 0)),
            *per_view_specs,
            full(HDH, DH),
            full(DH, 1),
            full(HDH, DH),
            full(DH, 1),
            full(N, HDH),
            full(1, 1),
        ],
        out_specs=[
            full(N, HDH),
            full(3, N, HDH),
        ],
        out_shape=[
            jax.ShapeDtypeStruct((N, HDH), jnp.float32),
            jax.ShapeDtypeStruct((3, N, HDH), jnp.float32),
        ],
        scratch_shapes=[
            pltpu.VMEM((3, H, NU, BU, DEP), jnp.bfloat16),
            pltpu.VMEM((3, H, NU, BU, DEP), jnp.bfloat16),
            pltpu.VMEM((3, NU, BU, H), jnp.bfloat16),
            pltpu.VMEM((3, H, N), jnp.bfloat16),
            pltpu.VMEM((3, H, N), jnp.float32),
            pltpu.VMEM((3, H, N), jnp.float32),
            pltpu.VMEM((3, H, N), jnp.float32),
            pltpu.VMEM((3, H, DEP, N), jnp.float32),
            pltpu.VMEM((3, H, DEP, N), jnp.float32),
        ],
    )(feature, s_adj, t_adj, poi_adj,
      sW, _block_diag_attn(s_al), _block_diag_attn(s_ar), s_b.reshape(HDH, 1),
      tW, _block_diag_attn(t_al), _block_diag_attn(t_ar), t_b.reshape(HDH, 1),
      pW, _block_diag_attn(p_al), _block_diag_attn(p_ar), p_b.reshape(HDH, 1),
      qW, qb.reshape(DH, 1), kW, kb.reshape(DH, 1),
      mvW.reshape(N, HDH), mvb.reshape(1, 1))

    return (mv_out, result)


# speedup vs baseline: 2.0571x; 1.0640x over previous
import jax
import jax.numpy as jnp
import numpy as np
from jax import lax
from jax.experimental import pallas as pl
from jax.experimental.pallas import tpu as pltpu

N = 1024
DIN = 256
H = 8
DH = 32
HDH = H * DH
DE = DH + 1
DEP = 40
NEG_SLOPE = 0.2
ALPHA = 0.8
BETA = 0.5

BU = 256
NU = N // BU


def _leaky(x):
    return jnp.where(x > 0, x, NEG_SLOPE * x)


def _fused_kernel(feat_ref,
                  adj0_ref, adj1_ref, adj2_ref,
                  W0_ref, W1_ref, W2_ref,
                  aa_ref, bb_ref,
                  qW_ref, qkb_ref, kW_ref, mvWT_ref, mvb_ref,
                  mvT_ref, resT_ref,
                  rhs1_s, rhs2_s, elb_s, nerT_s, B1_s, B2_s, coefT_s,
                  numA_s, numB_s):
    ui = pl.program_id(0)
    adj_refs = (adj0_ref, adj1_ref, adj2_ref)
    W_refs = (W0_ref, W1_ref, W2_ref)
    dnT = (((0,), (1,)), ((), ()))
    dn0 = (((0,), (0,)), ((), ()))

    @pl.when(ui == 0)
    def _setup():
        feat = feat_ref[...]
        ones_col = jnp.ones((N, 1), dtype=jnp.float32)
        for v in range(3):
            h = jnp.dot(feat, W_refs[v][...], preferred_element_type=jnp.float32)
            el = jnp.dot(h, aa_ref[2 * v], preferred_element_type=jnp.float32)
            elT = lax.dot_general(aa_ref[2 * v], h, dnT,
                                  preferred_element_type=jnp.float32)
            erT = lax.dot_general(aa_ref[2 * v + 1], h, dnT,
                                  preferred_element_type=jnp.float32)
            elmax_row = jnp.max(el, axis=0, keepdims=True)
            elmax_col = jnp.max(elT, axis=1, keepdims=True)
            MT = _leaky(elmax_col + erT)
            A1 = jnp.exp(el - elmax_row)
            A2 = jnp.exp(NEG_SLOPE * (el - elmax_row))
            B1_s[v] = jnp.exp(erT + elmax_col - MT)
            B2_s[v] = jnp.exp(NEG_SLOPE * (erT + elmax_col) - MT)
            coefT_s[v] = jnp.exp(_leaky(elT + erT) - MT)
            for uu in range(NU):
                sl = slice(uu * BU, (uu + 1) * BU)
                elb_s[v, uu] = el[sl, :].astype(jnp.bfloat16)
            nerT_s[v] = (-erT).astype(jnp.bfloat16)
            zpad = jnp.zeros((N, DEP - DE), dtype=jnp.bfloat16)
            for hh in range(H):
                hx = jnp.concatenate(
                    [h[:, hh * DH:(hh + 1) * DH], ones_col], axis=1)
                r1 = jnp.concatenate(
                    [(A1[:, hh:hh + 1] * hx).astype(jnp.bfloat16), zpad], axis=1)
                r2 = jnp.concatenate(
                    [(A2[:, hh:hh + 1] * hx).astype(jnp.bfloat16), zpad], axis=1)
                for uu in range(NU):
                    sl = slice(uu * BU, (uu + 1) * BU)
                    rhs1_s[v, hh, uu] = r1[sl, :]
                    rhs2_s[v, hh, uu] = r2[sl, :]
        numA_s[...] = jnp.zeros_like(numA_s)
        numB_s[...] = jnp.zeros_like(numB_s)

    rows = lax.broadcasted_iota(jnp.int32, (BU, N), 0) + ui * BU
    cols = lax.broadcasted_iota(jnp.int32, (BU, N), 1)
    offdiag = rows != cols
    for v in range(3):
        adjc = jnp.where(offdiag, jnp.ceil(adj_refs[v][...]),
                         0.0).astype(jnp.bfloat16)
        zero = jnp.zeros_like(adjc)
        for hh in range(H):
            cond = elb_s[v, ui, :, hh:hh + 1] >= nerT_s[v, hh:hh + 1, :]
            m1 = jnp.where(cond, adjc, zero)
            m2 = adjc - m1
            numA_s[v, hh] += lax.dot_general(
                rhs1_s[v, hh, ui], m1, dn0,
                preferred_element_type=jnp.float32)
            numB_s[v, hh] += lax.dot_general(
                rhs2_s[v, hh, ui], m2, dn0,
                preferred_element_type=jnp.float32)

    @pl.when(ui == NU - 1)
    def _finish():
        feat = feat_ref[...]
        d_k = jnp.sqrt(jnp.float32(DH * N))
        qW = qW_ref[...]
        kW = kW_ref[...]
        mvW = mvWT_ref[...]
        diag_mask = (lax.broadcasted_iota(jnp.int32, (DH, DH), 0)
                     == lax.broadcasted_iota(jnp.int32, (DH, DH), 1)
                     ).astype(jnp.float32)
        views = []
        logits = []
        gs = []
        for v in range(3):
            hT = lax.dot_general(W_refs[v][...], feat, dnT,
                                 preferred_element_type=jnp.float32)
            rows = []
            for hh in range(H):
                b1r = B1_s[v, hh:hh + 1, :]
                b2r = B2_s[v, hh:hh + 1, :]
                cfr = coefT_s[v, hh:hh + 1, :]
                hsl = hT[hh * DH:(hh + 1) * DH, :]
                num = (b1r * numA_s[v, hh, 0:DH, :]
                       + b2r * numB_s[v, hh, 0:DH, :] + cfr * hsl)
                den = (b1r * numA_s[v, hh, DH:DE, :]
                       + b2r * numB_s[v, hh, DH:DE, :] + cfr)
                o = num / den + bb_ref[v, hh * DH:(hh + 1) * DH, :]
                rows.append(jnp.maximum(o, 0.0))
            sv = jnp.concatenate(rows, axis=0)
            views.append(sv)
            Qv = lax.dot_general(qW, sv, dn0,
                                 preferred_element_type=jnp.float32) + qkb_ref[0]
            Kv = lax.dot_general(kW, sv, dn0,
                                 preferred_element_type=jnp.float32) + qkb_ref[1]
            logits.append(jnp.sum(Qv * Kv) / d_k)
            tr = jnp.float32(0.0)
            for hh in range(H):
                blk = lax.dot_general(
                    sv[hh * DH:(hh + 1) * DH, :], mvW[:, hh * DH:(hh + 1) * DH],
                    (((1,), (0,)), ((), ())), preferred_element_type=jnp.float32)
                tr = tr + jnp.sum(blk * diag_mask)
            gs.append(tr)

        m = jnp.maximum(jnp.maximum(logits[0], logits[1]), logits[2])
        ex = [jnp.exp(l - m) for l in logits]
        tot = ex[0] + ex[1] + ex[2]
        mvb = mvb_ref[0, 0]
        c = [ALPHA * (e / tot) + (1.0 - ALPHA) for e in ex]
        omega = [jax.nn.sigmoid(c[v] * gs[v] + mvb) for v in range(3)]
        mvT = (omega[0] * c[0] * views[0] + omega[1] * c[1] * views[1]
               + omega[2] * c[2] * views[2])
        mvT_ref[...] = mvT.T
        for v in range(3):
            resT_ref[v] = (BETA * c[v] * views[v] + (1.0 - BETA) * mvT).T


_BD_MASK = np.repeat(np.eye(H, dtype=np.float32), DH, axis=0)


@jax.jit
def kernel(feature, s_adj, t_adj, poi_adj,
           sW, s_al, s_ar, s_b,
           tW, t_al, t_ar, t_b,
           pW, p_al, p_ar, p_b,
           qW, qb, kW, kb, mvW, mvb):
    full = lambda *shape: pl.BlockSpec(shape, lambda ui: (0,) * len(shape))
    aa = (jnp.stack([s_al, s_ar, t_al, t_ar, p_al, p_ar]
                    ).reshape(6, HDH, 1) * _BD_MASK)
    bb = jnp.stack([s_b, t_b, p_b]).reshape(3, HDH, 1)
    qkb = jnp.stack([qb, kb]).reshape(2, DH, 1)

    mv_out, result = pl.pallas_call(
        _fused_kernel,
        grid=(NU,),
        in_specs=[
            full(N, DIN),
            pl.BlockSpec((BU, N), lambda ui: (ui, 0)),
            pl.BlockSpec((BU, N), lambda ui: (ui, 0)),
            pl.BlockSpec((BU, N), lambda ui: (ui, 0)),
            full(DIN, HDH),
            full(DIN, HDH),
            full(DIN, HDH),
            full(6, HDH, H),
            full(3, HDH, 1),
            full(HDH, DH),
            full(2, DH, 1),
            full(HDH, DH),
            full(N, HDH),
            full(1, 1),
        ],
        out_specs=[
            full(N, HDH),
            full(3, N, HDH),
        ],
        out_shape=[
            jax.ShapeDtypeStruct((N, HDH), jnp.float32),
            jax.ShapeDtypeStruct((3, N, HDH), jnp.float32),
        ],
        scratch_shapes=[
            pltpu.VMEM((3, H, NU, BU, DEP), jnp.bfloat16),
            pltpu.VMEM((3, H, NU, BU, DEP), jnp.bfloat16),
            pltpu.VMEM((3, NU, BU, H), jnp.bfloat16),
            pltpu.VMEM((3, H, N), jnp.bfloat16),
            pltpu.VMEM((3, H, N), jnp.float32),
            pltpu.VMEM((3, H, N), jnp.float32),
            pltpu.VMEM((3, H, N), jnp.float32),
            pltpu.VMEM((3, H, DEP, N), jnp.float32),
            pltpu.VMEM((3, H, DEP, N), jnp.float32),
        ],
    )(feature, s_adj, t_adj, poi_adj,
      sW, tW, pW, aa, bb, qW, qkb, kW,
      mvW.reshape(N, HDH), mvb.reshape(1, 1))

    return (mv_out, result)
